# baseline (speedup 1.0000x reference)
import functools

import jax
import jax.numpy as jnp
from jax import lax
from jax.experimental import pallas as pl
from jax.experimental.pallas import tpu as pltpu
from jax.experimental.pallas import tpu_sc as plsc

NC = 2
NS = 16
NW = NC * NS

B_TOK, L_TOK = 4096, 200
D = 64
VOCAB = 1000000
NP_FULL = 7812
SROWS = NP_FULL * 64 + 32
N_GROUPS = (L_TOK // 8) * (B_TOK // 128)
GROUPS_PER_W = N_GROUPS // NW


def _transpose_body(wt_hbm, wtail_hbm, s_hbm, pb0, pb1, tb0, tb1, *sems):
    gsem = sems[0:2]
    ssem = sems[2:4]
    pbufs = (pb0, pb1)
    tbufs = (tb0, tb1)
    c = lax.axis_index("c")
    s = lax.axis_index("s")
    wid = s * NC + c
    iota = lax.iota(jnp.int32, 16)

    def colbase(k):
        return pl.multiple_of((wid + NW * k) * 128, 128)

    def start_in(k, b):
        pltpu.async_copy(
            wt_hbm.at[pl.ds(0, D), pl.ds(colbase(k), 128)], pbufs[b], gsem[b]
        )

    def wait_in(k, b):
        pltpu.make_async_copy(
            wt_hbm.at[pl.ds(0, D), pl.ds(colbase(k), 128)], pbufs[b], gsem[b]
        ).wait()

    def start_out(k, b):
        pltpu.async_copy(
            tbufs[b], s_hbm.at[pl.ds(pl.multiple_of(colbase(k) // 2, 8), 64)],
            ssem[b],
        )

    def wait_out(k, b):
        pltpu.make_async_copy(
            tbufs[b], s_hbm.at[pl.ds(pl.multiple_of(colbase(k) // 2, 8), 64)],
            ssem[b],
        ).wait()

    def transpose(b):
        pbuf = pbufs[b]
        tbuf = tbufs[b]

        def trans_q(q, carry):
            c0 = 2 * q
            for h in range(8):
                row_idx = iota + (h % 4) * 16
                col_idx = jnp.full((16,), c0 + (1 if h >= 4 else 0), jnp.int32)
                v = plsc.load_gather(pbuf, [row_idx, col_idx])
                tbuf[q, pl.ds(h * 16, 16)] = v
            return carry

        lax.fori_loop(0, 64, trans_q, 0)

    n_full = NP_FULL // NW
    @pl.when(wid == NW - 1)
    def _():
        pltpu.sync_copy(wtail_hbm, s_hbm.at[pl.ds(NP_FULL * 64, 32)])

    start_in(0, 0)

    def step(k, carry):
        for b in range(2):
            kk = 2 * k + b
            wait_in(kk, b)

            @pl.when(kk + 1 < n_full)
            def _():
                start_in(kk + 1, 1 - b)

            @pl.when(kk >= 2)
            def _():
                wait_out(kk - 2, b)

            transpose(b)
            start_out(kk, b)
        return carry

    lax.fori_loop(0, n_full // 2, step, 0)
    wait_out(n_full - 2, 0)
    wait_out(n_full - 1, 1)

    @pl.when(wid + NW * n_full < NP_FULL)
    def _():
        start_in(n_full, 0)
        wait_in(n_full, 0)
        transpose(0)
        start_out(n_full, 0)
        wait_out(n_full, 0)


def _gather_body(s_hbm, tid_hbm, out_hbm,
                 ivmem, pidx0, pidx1, par0, par1, rv0, rv1, qv0, qv1, *sems):
    isem = sems[0]
    gsem = sems[1:3]
    osem = sems[3:5]
    pidxs = (pidx0, pidx1)
    pars = (par0, par1)
    rvs = (rv0, rv1)
    qvs = (qv0, qv1)
    c = lax.axis_index("c")
    s = lax.axis_index("s")
    wid = s * NC + c
    iota = lax.iota(jnp.int32, 16)

    def prep(r, b):
        for cc in range(8):
            v = ivmem[r, pl.ds(cc * 16, 16)]
            pidxs[b][pl.ds(cc * 16, 16)] = lax.shift_right_logical(v, 1)
            pars[b][pl.ds(cc * 16, 16)] = lax.shift_left(
                lax.bitwise_and(v, 1), 6
            )

    def start_gather(b):
        pltpu.async_copy(s_hbm.at[pidxs[b]], rvs[b], gsem[b])

    def wait_gather(b):
        pltpu.make_async_copy(s_hbm.at[pidxs[b]], rvs[b], gsem[b]).wait()

    def build_q(b):
        rv = rvs[b]
        qv = qvs[b]
        par = pars[b]

        def per_dt(dt, carry):
            for cc in range(8):
                p64 = par[pl.ds(cc * 16, 16)]
                rowi = iota + cc * 16
                base = p64 + dt * 8
                for r in range(8):
                    v = plsc.load_gather(rv, [rowi, base + r])
                    qv[dt, r, pl.ds(cc * 16, 16)] = v
            return carry

        lax.fori_loop(0, 8, per_dt, 0)

    def start_out(g, r, b):
        gid = wid * GROUPS_PER_W + g
        lt = gid // 32
        bt = gid % 32
        pltpu.async_copy(qvs[b], out_hbm.at[lt * 8 + r, pl.ds(0, 8), bt], osem[b])

    def wait_out(g, r, b):
        gid = wid * GROUPS_PER_W + g
        lt = gid // 32
        bt = gid % 32
        pltpu.make_async_copy(
            qvs[b], out_hbm.at[lt * 8 + r, pl.ds(0, 8), bt], osem[b]
        ).wait()

    def group(g, carry):
        gid = wid * GROUPS_PER_W + g
        lt = gid // 32
        bt = gid % 32
        pltpu.sync_copy(
            tid_hbm.at[
                pl.ds(pl.multiple_of(lt * 8, 8), 8),
                pl.ds(pl.multiple_of(bt * 128, 128), 128),
            ],
            ivmem,
        )
        prep(0, 0)
        start_gather(0)
        for rr in range(4):
            for b in range(2):
                r = 2 * rr + b
                wait_gather(b)
                if r + 1 < 8:
                    prep(r + 1, 1 - b)
                    start_gather(1 - b)
                if r >= 2:
                    wait_out(g, r - 2, b)
                build_q(b)
                start_out(g, r, b)
        wait_out(g, 6, 0)
        wait_out(g, 7, 1)
        return carry

    lax.fori_loop(0, GROUPS_PER_W, group, 0)


@jax.jit
def _embed(token_ids, weight):
    wt = weight.T
    tid = token_ids.T
    wtail = weight[NP_FULL * 128:].reshape(32, 128)
    mesh = plsc.VectorSubcoreMesh(core_axis_name="c", subcore_axis_name="s")
    params = pltpu.CompilerParams(
        use_tc_tiling_on_sc=True, needs_layout_passes=False
    )

    s_tab = pl.kernel(
        _transpose_body,
        out_type=jax.ShapeDtypeStruct((SROWS, 128), jnp.float32),
        mesh=mesh,
        scratch_types=[
            pltpu.VMEM((D, 128), jnp.float32),
            pltpu.VMEM((D, 128), jnp.float32),
            pltpu.VMEM((D, 128), jnp.float32),
            pltpu.VMEM((D, 128), jnp.float32),
        ] + [pltpu.SemaphoreType.DMA] * 4,
        compiler_params=params,
    )(wt, wtail)

    phys = pl.kernel(
        _gather_body,
        out_type=jax.ShapeDtypeStruct((L_TOK, 8, 32, 8, 128), jnp.float32),
        mesh=mesh,
        scratch_types=[
            pltpu.VMEM((8, 128), jnp.int32),
            pltpu.VMEM((128,), jnp.int32),
            pltpu.VMEM((128,), jnp.int32),
            pltpu.VMEM((128,), jnp.int32),
            pltpu.VMEM((128,), jnp.int32),
            pltpu.VMEM((128, 128), jnp.float32),
            pltpu.VMEM((128, 128), jnp.float32),
            pltpu.VMEM((8, 8, 128), jnp.float32),
            pltpu.VMEM((8, 8, 128), jnp.float32),
        ] + [pltpu.SemaphoreType.DMA] * 5,
        compiler_params=params,
    )(s_tab, tid)

    return phys.transpose(2, 4, 0, 1, 3).reshape(B_TOK, L_TOK, D)


def kernel(token_ids, weight):
    return _embed(token_ids, weight)


# speedup vs baseline: 4.3564x; 4.3564x over previous
import functools

import jax
import jax.numpy as jnp
from jax import lax
from jax.experimental import pallas as pl
from jax.experimental.pallas import tpu as pltpu
from jax.experimental.pallas import tpu_sc as plsc

NC = 2
NS = 16
NW = NC * NS

B_TOK, L_TOK = 4096, 200
D = 64
VOCAB = 1000000
NP_FULL = 7812
SROWS = NP_FULL * 64 + 32
N_GROUPS = (L_TOK // 8) * (B_TOK // 128)
GROUPS_PER_W = N_GROUPS // NW


def _transpose_body(wt_hbm, wtail_hbm, s_hbm, pb0, pb1, tb0, tb1, *sems):
    gsem = sems[0:2]
    ssem = sems[2:4]
    pbufs = (pb0, pb1)
    tbufs = (tb0, tb1)
    c = lax.axis_index("c")
    s = lax.axis_index("s")
    wid = s * NC + c
    iota = lax.iota(jnp.int32, 16)

    def colbase(k):
        return pl.multiple_of((wid + NW * k) * 128, 128)

    def start_in(k, b):
        pltpu.async_copy(
            wt_hbm.at[pl.ds(0, D), pl.ds(colbase(k), 128)], pbufs[b], gsem[b]
        )

    def wait_in(k, b):
        pltpu.make_async_copy(
            wt_hbm.at[pl.ds(0, D), pl.ds(colbase(k), 128)], pbufs[b], gsem[b]
        ).wait()

    def start_out(k, b):
        pltpu.async_copy(
            tbufs[b], s_hbm.at[pl.ds(pl.multiple_of(colbase(k) // 2, 8), 64)],
            ssem[b],
        )

    def wait_out(k, b):
        pltpu.make_async_copy(
            tbufs[b], s_hbm.at[pl.ds(pl.multiple_of(colbase(k) // 2, 8), 64)],
            ssem[b],
        ).wait()

    def transpose(b):
        pbuf = pbufs[b]
        tbuf = tbufs[b]

        @functools.partial(plsc.parallel_loop, 0, 64, unroll=4)
        def _(q):
            cidx0 = jnp.full((16,), 2 * q, jnp.int32)
            cidx1 = cidx0 + 1
            for h in range(8):
                row_idx = iota + (h % 4) * 16
                v = plsc.load_gather(
                    pbuf, [row_idx, cidx1 if h >= 4 else cidx0]
                )
                tbuf[q, pl.ds(h * 16, 16)] = v

    n_full = NP_FULL // NW
    @pl.when(wid == NW - 1)
    def _():
        pltpu.sync_copy(wtail_hbm, s_hbm.at[pl.ds(NP_FULL * 64, 32)])

    start_in(0, 0)

    def step(k, carry):
        for b in range(2):
            kk = 2 * k + b
            wait_in(kk, b)

            @pl.when(kk + 1 < n_full)
            def _():
                start_in(kk + 1, 1 - b)

            @pl.when(kk >= 2)
            def _():
                wait_out(kk - 2, b)

            transpose(b)
            start_out(kk, b)
        return carry

    lax.fori_loop(0, n_full // 2, step, 0)
    wait_out(n_full - 2, 0)
    wait_out(n_full - 1, 1)

    @pl.when(wid + NW * n_full < NP_FULL)
    def _():
        start_in(n_full, 0)
        wait_in(n_full, 0)
        transpose(0)
        start_out(n_full, 0)
        wait_out(n_full, 0)


def _gather_body(s_hbm, tid_hbm, out_hbm,
                 ivmem, pidx0, pidx1, par0, par1, rv0, rv1, qv0, qv1, *sems):
    isem = sems[0]
    gsem = sems[1:3]
    osem = sems[3:5]
    pidxs = (pidx0, pidx1)
    pars = (par0, par1)
    rvs = (rv0, rv1)
    qvs = (qv0, qv1)
    c = lax.axis_index("c")
    s = lax.axis_index("s")
    wid = s * NC + c
    iota = lax.iota(jnp.int32, 16)

    def prep(r, b):
        for cc in range(8):
            v = ivmem[r, pl.ds(cc * 16, 16)]
            pidxs[b][pl.ds(cc * 16, 16)] = lax.shift_right_logical(v, 1)
            pars[b][pl.ds(cc * 16, 16)] = lax.shift_left(
                lax.bitwise_and(v, 1), 6
            )

    def start_gather(b):
        pltpu.async_copy(s_hbm.at[pidxs[b]], rvs[b], gsem[b])

    def wait_gather(b):
        pltpu.make_async_copy(s_hbm.at[pidxs[b]], rvs[b], gsem[b]).wait()

    def build_q(b):
        rv = rvs[b]
        qv = qvs[b]
        par = pars[b]

        @functools.partial(plsc.parallel_loop, 0, 8, unroll=2)
        def _(dt):
            for cc in range(8):
                p64 = par[pl.ds(cc * 16, 16)]
                rowi = iota + cc * 16
                base = p64 + dt * 8
                for r in range(8):
                    v = plsc.load_gather(rv, [rowi, base + r])
                    qv[dt, r, pl.ds(cc * 16, 16)] = v

    def start_out(g, r, b):
        gid = wid * GROUPS_PER_W + g
        lt = gid // 32
        bt = gid % 32
        pltpu.async_copy(qvs[b], out_hbm.at[lt * 8 + r, pl.ds(0, 8), bt], osem[b])

    def wait_out(g, r, b):
        gid = wid * GROUPS_PER_W + g
        lt = gid // 32
        bt = gid % 32
        pltpu.make_async_copy(
            qvs[b], out_hbm.at[lt * 8 + r, pl.ds(0, 8), bt], osem[b]
        ).wait()

    def group(g, carry):
        gid = wid * GROUPS_PER_W + g
        lt = gid // 32
        bt = gid % 32
        pltpu.sync_copy(
            tid_hbm.at[
                pl.ds(pl.multiple_of(lt * 8, 8), 8),
                pl.ds(pl.multiple_of(bt * 128, 128), 128),
            ],
            ivmem,
        )
        prep(0, 0)
        start_gather(0)
        for rr in range(4):
            for b in range(2):
                r = 2 * rr + b
                wait_gather(b)
                if r + 1 < 8:
                    prep(r + 1, 1 - b)
                    start_gather(1 - b)
                if r >= 2:
                    wait_out(g, r - 2, b)
                build_q(b)
                start_out(g, r, b)
        wait_out(g, 6, 0)
        wait_out(g, 7, 1)
        return carry

    lax.fori_loop(0, GROUPS_PER_W, group, 0)


@jax.jit
def _embed(token_ids, weight):
    wt = weight.T
    tid = token_ids.T
    wtail = weight[NP_FULL * 128:].reshape(32, 128)
    mesh = plsc.VectorSubcoreMesh(core_axis_name="c", subcore_axis_name="s")
    params = pltpu.CompilerParams(
        use_tc_tiling_on_sc=True, needs_layout_passes=False
    )

    s_tab = pl.kernel(
        _transpose_body,
        out_type=jax.ShapeDtypeStruct((SROWS, 128), jnp.float32),
        mesh=mesh,
        scratch_types=[
            pltpu.VMEM((D, 128), jnp.float32),
            pltpu.VMEM((D, 128), jnp.float32),
            pltpu.VMEM((D, 128), jnp.float32),
            pltpu.VMEM((D, 128), jnp.float32),
        ] + [pltpu.SemaphoreType.DMA] * 4,
        compiler_params=params,
    )(wt, wtail)

    phys = pl.kernel(
        _gather_body,
        out_type=jax.ShapeDtypeStruct((L_TOK, 8, 32, 8, 128), jnp.float32),
        mesh=mesh,
        scratch_types=[
            pltpu.VMEM((8, 128), jnp.int32),
            pltpu.VMEM((128,), jnp.int32),
            pltpu.VMEM((128,), jnp.int32),
            pltpu.VMEM((128,), jnp.int32),
            pltpu.VMEM((128,), jnp.int32),
            pltpu.VMEM((128, 128), jnp.float32),
            pltpu.VMEM((128, 128), jnp.float32),
            pltpu.VMEM((8, 8, 128), jnp.float32),
            pltpu.VMEM((8, 8, 128), jnp.float32),
        ] + [pltpu.SemaphoreType.DMA] * 5,
        compiler_params=params,
    )(s_tab, tid)

    return phys.transpose(2, 4, 0, 1, 3).reshape(B_TOK, L_TOK, D)


def kernel(token_ids, weight):
    return _embed(token_ids, weight)
